# baseline (speedup 1.0000x reference)
import functools

import jax
import jax.numpy as jnp
from jax import lax
from jax.experimental import pallas as pl
from jax.experimental.pallas import tpu as pltpu
from jax.experimental.pallas import tpu_sc as plsc

B, S, D, E, TOPK = 128, 577, 1024, 16, 2
NC, NS, L = 2, 16, 16
NW = NC * NS
RPW = B // NW
CHUNKS = D // L
EGRP = 8
UNROLL = 2

_mesh = plsc.VectorSubcoreMesh(core_axis_name="c", subcore_axis_name="s")


@functools.partial(
    pl.kernel,
    out_type=[
        jax.ShapeDtypeStruct((B * TOPK,), jnp.float32),
        jax.ShapeDtypeStruct((B * TOPK,), jnp.int32),
    ],
    mesh=_mesh,
    compiler_params=pltpu.CompilerParams(
        needs_layout_passes=False,
        skip_device_barrier=True,
        disable_bounds_checks=True,
    ),
    scratch_types=[
        pltpu.VMEM((RPW, D), jnp.float32),
        pltpu.VMEM((E, D), jnp.float32),
        pltpu.VMEM((L,), jnp.float32),
        pltpu.VMEM((L,), jnp.float32),
        pltpu.VMEM((L,), jnp.int32),
    ],
)
def _router_kernel(cls_hbm, w_hbm, b_hbm, out_w_hbm, out_i_hbm,
                   x_ref, w_ref, b_ref, ow_ref, oi_ref):
    wid = lax.axis_index("s") * NC + lax.axis_index("c")
    base = wid * RPW

    pltpu.sync_copy(w_hbm, w_ref)
    pltpu.sync_copy(b_hbm, b_ref)
    pltpu.sync_copy(cls_hbm.at[pl.ds(base, RPW)], x_ref)

    lanes = lax.iota(jnp.int32, L)
    b_vec = b_ref[...]
    zero = jnp.zeros((L,), jnp.float32)
    lvecs = [zero for _ in range(RPW)]

    for grp in range(E // EGRP):
        e0 = grp * EGRP

        def body(c, accs, e0=e0):
            new = list(accs)
            for u in range(UNROLL):
                off = (c * UNROLL + u) * L
                xs = [x_ref[r, pl.ds(off, L)] for r in range(RPW)]
                for ei in range(EGRP):
                    wv = w_ref[e0 + ei, pl.ds(off, L)]
                    for r in range(RPW):
                        k = ei * RPW + r
                        new[k] = new[k] + xs[r] * wv
            return tuple(new)

        accs = lax.fori_loop(0, CHUNKS // UNROLL, body,
                             tuple(zero for _ in range(EGRP * RPW)))
        for ei in range(EGRP):
            for r in range(RPW):
                s = jnp.sum(accs[ei * RPW + r])
                lvecs[r] = jnp.where(lanes == (e0 + ei), s, lvecs[r])

    neg = jnp.float32(-3.0e38)
    ow = zero
    oi = jnp.zeros((L,), jnp.int32)
    for r in range(RPW):
        lv = lvecs[r] + b_vec
        m1 = jnp.max(lv)
        i1 = jnp.min(jnp.where(lv == m1, lanes, E))
        masked = jnp.where(lanes == i1, neg, lv)
        m2 = jnp.max(masked)
        i2 = jnp.min(jnp.where(masked == m2, lanes, E))
        t = jnp.exp(jnp.full((L,), m2 - m1, jnp.float32))
        w1 = 1.0 / (1.0 + t)
        w2 = t / (1.0 + t)
        ow = jnp.where(lanes == 2 * r, w1, ow)
        ow = jnp.where(lanes == 2 * r + 1, w2, ow)
        oi = jnp.where(lanes == 2 * r, i1, oi)
        oi = jnp.where(lanes == 2 * r + 1, i2, oi)

    ow_ref[...] = ow
    oi_ref[...] = oi
    pltpu.sync_copy(ow_ref.at[pl.ds(0, TOPK * RPW)],
                    out_w_hbm.at[pl.ds(TOPK * base, TOPK * RPW)])
    pltpu.sync_copy(oi_ref.at[pl.ds(0, TOPK * RPW)],
                    out_i_hbm.at[pl.ds(TOPK * base, TOPK * RPW)])


def _round_to_bf16(x):
    u = lax.bitcast_convert_type(x, jnp.uint32)
    r = (u + jnp.uint32(0x7FFF) + ((u >> 16) & jnp.uint32(1))) & jnp.uint32(
        0xFFFF0000
    )
    return lax.bitcast_convert_type(r, jnp.float32)


def kernel(vision_features, W, b):
    cls_tok = _round_to_bf16(vision_features[:, 0])
    w_r = _round_to_bf16(W)
    ow, oi = _router_kernel(cls_tok, w_r, b)
    return ow.reshape(B, TOPK), oi.reshape(B, TOPK)


# speedup vs baseline: 1.0998x; 1.0998x over previous
import functools

import jax
import jax.numpy as jnp
from jax import lax
from jax.experimental import pallas as pl
from jax.experimental.pallas import tpu as pltpu
from jax.experimental.pallas import tpu_sc as plsc

B, S, D, E, TOPK = 128, 577, 1024, 16, 2
NC, NS, L = 2, 16, 16
NW = NC * NS
RPW = B // NW
CHUNKS = D // L
EGRP = 8
UNROLL = 1

_mesh = plsc.VectorSubcoreMesh(core_axis_name="c", subcore_axis_name="s")


@functools.partial(
    pl.kernel,
    out_type=[
        jax.ShapeDtypeStruct((B * TOPK,), jnp.float32),
        jax.ShapeDtypeStruct((B * TOPK,), jnp.int32),
    ],
    mesh=_mesh,
    compiler_params=pltpu.CompilerParams(
        needs_layout_passes=False,
        skip_device_barrier=True,
        disable_bounds_checks=True,
    ),
    scratch_types=[
        pltpu.VMEM((RPW, D), jnp.float32),
        pltpu.VMEM((E, D), jnp.float32),
        pltpu.VMEM((L,), jnp.float32),
        pltpu.VMEM((L,), jnp.float32),
        pltpu.VMEM((L,), jnp.int32),
    ],
)
def _router_kernel(packed_hbm, b_hbm, out_w_hbm, out_i_hbm,
                   x_ref, w_ref, b_ref, ow_ref, oi_ref):
    wid = lax.axis_index("s") * NC + lax.axis_index("c")
    base = wid * RPW

    pltpu.sync_copy(packed_hbm.at[pl.ds(B, E)], w_ref)
    pltpu.sync_copy(b_hbm, b_ref)
    pltpu.sync_copy(packed_hbm.at[pl.ds(base, RPW)], x_ref)

    lanes = lax.iota(jnp.int32, L)
    b_vec = b_ref[...]
    zero = jnp.zeros((L,), jnp.float32)
    lvecs = [zero for _ in range(RPW)]

    for grp in range(E // EGRP):
        e0 = grp * EGRP

        def body(c, accs, e0=e0):
            new = list(accs)
            for u in range(UNROLL):
                off = (c * UNROLL + u) * L
                xs = [x_ref[r, pl.ds(off, L)] for r in range(RPW)]
                for ei in range(EGRP):
                    wv = w_ref[e0 + ei, pl.ds(off, L)]
                    for r in range(RPW):
                        k = ei * RPW + r
                        new[k] = new[k] + xs[r] * wv
            return tuple(new)

        accs = lax.fori_loop(0, CHUNKS // UNROLL, body,
                             tuple(zero for _ in range(EGRP * RPW)))
        for ei in range(EGRP):
            for r in range(RPW):
                s = jnp.sum(accs[ei * RPW + r])
                lvecs[r] = jnp.where(lanes == (e0 + ei), s, lvecs[r])

    neg = jnp.float32(-3.0e38)
    ow = zero
    oi = jnp.zeros((L,), jnp.int32)
    for r in range(RPW):
        lv = lvecs[r] + b_vec
        m1 = jnp.max(lv)
        i1 = jnp.min(jnp.where(lv == m1, lanes, E))
        masked = jnp.where(lanes == i1, neg, lv)
        m2 = jnp.max(masked)
        i2 = jnp.min(jnp.where(masked == m2, lanes, E))
        t = jnp.exp(jnp.full((L,), m2 - m1, jnp.float32))
        w1 = 1.0 / (1.0 + t)
        w2 = t / (1.0 + t)
        ow = jnp.where(lanes == 2 * r, w1, ow)
        ow = jnp.where(lanes == 2 * r + 1, w2, ow)
        oi = jnp.where(lanes == 2 * r, i1, oi)
        oi = jnp.where(lanes == 2 * r + 1, i2, oi)

    ow_ref[...] = ow
    oi_ref[...] = oi
    pltpu.sync_copy(ow_ref.at[pl.ds(0, TOPK * RPW)],
                    out_w_hbm.at[pl.ds(TOPK * base, TOPK * RPW)])
    pltpu.sync_copy(oi_ref.at[pl.ds(0, TOPK * RPW)],
                    out_i_hbm.at[pl.ds(TOPK * base, TOPK * RPW)])


def _round_to_bf16(x):
    u = lax.bitcast_convert_type(x, jnp.uint32)
    r = (u + jnp.uint32(0x7FFF) + ((u >> 16) & jnp.uint32(1))) & jnp.uint32(
        0xFFFF0000
    )
    return lax.bitcast_convert_type(r, jnp.float32)


def kernel(vision_features, W, b):
    packed = _round_to_bf16(
        jnp.concatenate([vision_features[:, 0], W], axis=0))
    ow, oi = _router_kernel(packed, b)
    return ow.reshape(B, TOPK), oi.reshape(B, TOPK)
